# baseline (speedup 1.0000x reference)
import functools

import jax
import jax.numpy as jnp
from jax import lax
from jax.experimental import pallas as pl
from jax.experimental.pallas import tpu as pltpu
from jax.experimental.pallas import tpu_sc as plsc

_VOCAB = 128
_D = 32
_NC = 2
_NS = 16
_NW = _NC * _NS
_R = 128
_CR = 8
_H = _CR // 2


@functools.lru_cache(maxsize=None)
def _make_kernel(nrows: int):
  rows_w = nrows // _NW
  nch = rows_w // _CR
  assert nrows % _NW == 0 and rows_w % _CR == 0 and nch % 2 == 0

  mesh = plsc.VectorSubcoreMesh(core_axis_name="c", subcore_axis_name="s")

  @functools.partial(
      pl.kernel,
      out_type=jax.ShapeDtypeStruct((nrows, _R, _D), jnp.float32),
      mesh=mesh,
      compiler_params=pltpu.CompilerParams(
          needs_layout_passes=False, use_tc_tiling_on_sc=False),
      scratch_types=[
          pltpu.VMEM_SHARED((_VOCAB, _D), jnp.float32),
          pltpu.VMEM((_CR, _R), jnp.int32),
          pltpu.VMEM((_CR, _R), jnp.int32),
          pltpu.VMEM((_CR, _R, _D), jnp.float32),
          pltpu.VMEM((_CR, _R, _D), jnp.float32),
          pltpu.SemaphoreType.DMA,
          pltpu.SemaphoreType.DMA,
          pltpu.SemaphoreType.DMA,
          pltpu.SemaphoreType.DMA,
          pltpu.SemaphoreType.DMA,
          pltpu.SemaphoreType.DMA,
          pltpu.SemaphoreType.DMA,
      ],
  )
  def emb(x_hbm, table_hbm, out_hbm,
          table_sh, iv0, iv1, ov0, ov1,
          si0, si1, sg, soa0, sob0, soa1, sob1):
    wid = lax.axis_index("s") * _NC + lax.axis_index("c")
    w_base = wid * rows_w
    ivs = (iv0, iv1)
    ovs = (ov0, ov1)
    sis = (si0, si1)
    sos = ((soa0, sob0), (soa1, sob1))

    @pl.when(lax.axis_index("s") == 0)
    def _():
      pltpu.sync_copy(table_hbm, table_sh)
    plsc.subcore_barrier()

    for b in range(2):
      pltpu.async_copy(
          x_hbm.at[pl.ds(w_base + b * _CR, _CR)], ivs[b], sis[b])

    def outer(gi, carry):
      for b in range(2):
        g = gi * 2 + b
        iv, ov, si = ivs[b], ovs[b], sis[b]
        so = sos[b]
        base = w_base + g * _CR

        pltpu.make_async_copy(x_hbm.at[pl.ds(w_base, _CR)], iv, si).wait()

        @pl.when(gi > 0)
        def _():
          for h in range(2):
            pltpu.make_async_copy(
                ov.at[pl.ds(h * _H, _H)],
                out_hbm.at[pl.ds(w_base, _H)], so[h]).wait()

        gathers = [
            pltpu.async_copy(table_sh.at[iv.at[r]], ov.at[r], sg)
            for r in range(_CR)
        ]
        for hnd in gathers:
          hnd.wait()

        nxt = jnp.minimum(g + 2, nch - 1)
        pltpu.async_copy(x_hbm.at[pl.ds(w_base + nxt * _CR, _CR)], iv, si)
        for h in range(2):
          pltpu.async_copy(
              ov.at[pl.ds(h * _H, _H)],
              out_hbm.at[pl.ds(base + h * _H, _H)], so[h])
      return carry

    lax.fori_loop(0, nch // 2, outer, 0, unroll=False)

    for b in range(2):
      pltpu.make_async_copy(
          x_hbm.at[pl.ds(w_base, _CR)], ivs[b], sis[b]).wait()
      for h in range(2):
        pltpu.make_async_copy(
            ovs[b].at[pl.ds(h * _H, _H)],
            out_hbm.at[pl.ds(w_base, _H)], sos[b][h]).wait()

  return emb


def kernel(x, table):
  xf = x.reshape(-1, _R).astype(jnp.int32)
  out = _make_kernel(xf.shape[0])(xf, table)
  return out.reshape(*x.shape, _D)
